# baseline (speedup 1.0000x reference)
import functools

import jax
import jax.numpy as jnp
from jax import lax
from jax.experimental import pallas as pl
from jax.experimental.pallas import tpu as pltpu
from jax.experimental.pallas import tpu_sc as plsc

VOCAB = 1000000
EMB = 32
B = 4096
L = 50

NC = 2
NS = 16
NW = NC * NS

TOTAL_ROWS = 2 * B * L
ROWS_PER_W = TOTAL_ROWS // NW
CHUNK = 1600
NCHUNK = ROWS_PER_W // CHUNK


def _gather_body(table_hbm, idx_hbm, out_hbm, idx_v, rows_v, sem):
    wid = lax.axis_index("s") * NC + lax.axis_index("c")
    base = wid * ROWS_PER_W

    def chunk_step(c, _):
        off = base + c * CHUNK
        pltpu.sync_copy(idx_hbm.at[pl.ds(off, CHUNK)], idx_v)
        pltpu.async_copy(table_hbm.at[idx_v], rows_v, sem).wait()
        pltpu.sync_copy(rows_v, out_hbm.at[pl.ds(off, CHUNK)])
        return _

    lax.fori_loop(0, NCHUNK, chunk_step, 0)


@jax.jit
def _embed_gather(table, idx):
    mesh = plsc.VectorSubcoreMesh(core_axis_name="c", subcore_axis_name="s")
    return pl.kernel(
        _gather_body,
        out_type=jax.ShapeDtypeStruct((TOTAL_ROWS, EMB), jnp.float32),
        mesh=mesh,
        scratch_types=[
            pltpu.VMEM((CHUNK,), jnp.int32),
            pltpu.VMEM((CHUNK, EMB), jnp.float32),
            pltpu.SemaphoreType.DMA,
        ],
        compiler_params=pltpu.CompilerParams(use_tc_tiling_on_sc=False),
    )(table, idx)


def kernel(table, post, resp):
    idx = jnp.concatenate(
        [post.reshape(-1), resp.reshape(-1)]).astype(jnp.int32)
    rows = _embed_gather(table, idx)
    return rows.reshape(2, B, L, EMB)


# speedup vs baseline: 1.0024x; 1.0024x over previous
import functools

import jax
import jax.numpy as jnp
from jax import lax
from jax.experimental import pallas as pl
from jax.experimental.pallas import tpu as pltpu
from jax.experimental.pallas import tpu_sc as plsc

VOCAB = 1000000
EMB = 32
B = 4096
L = 50

NC = 2
NS = 16
NW = NC * NS

TOTAL_ROWS = 2 * B * L
ROWS_PER_W = TOTAL_ROWS // NW
CHUNK = 1280
NCHUNK = ROWS_PER_W // CHUNK
NBUF = 2


def _gather_body(table_hbm, idx_hbm, out_hbm,
                 idx_v0, idx_v1, rows_v0, rows_v1,
                 si0, si1, sg0, sg1, so0, so1):
    wid = lax.axis_index("s") * NC + lax.axis_index("c")
    base = wid * ROWS_PER_W
    ids = (idx_v0, idx_v1)
    rows = (rows_v0, rows_v1)
    si = (si0, si1)
    sg = (sg0, sg1)
    so = (so0, so1)

    def out_at(c):
        return out_hbm.at[pl.ds(base + c * CHUNK, CHUNK)]

    def idx_at(c):
        return idx_hbm.at[pl.ds(base + c * CHUNK, CHUNK)]

    for c in range(min(NBUF, NCHUNK)):
        pltpu.async_copy(idx_at(c), ids[c % NBUF], si[c % NBUF])

    for c in range(NCHUNK):
        b = c % NBUF
        pltpu.make_async_copy(idx_at(c), ids[b], si[b]).wait()
        if c >= NBUF:
            pltpu.make_async_copy(rows[b], out_at(c - NBUF), so[b]).wait()
        pltpu.async_copy(table_hbm.at[ids[b]], rows[b], sg[b]).wait()
        pltpu.async_copy(rows[b], out_at(c), so[b])
        if c + NBUF < NCHUNK:
            pltpu.async_copy(idx_at(c + NBUF), ids[b], si[b])

    for c in range(max(NCHUNK - NBUF, 0), NCHUNK):
        b = c % NBUF
        pltpu.make_async_copy(rows[b], out_at(c), so[b]).wait()


@jax.jit
def _embed_gather(table, idx):
    mesh = plsc.VectorSubcoreMesh(core_axis_name="c", subcore_axis_name="s")
    return pl.kernel(
        _gather_body,
        out_type=jax.ShapeDtypeStruct((TOTAL_ROWS, EMB), jnp.float32),
        mesh=mesh,
        scratch_types=[
            pltpu.VMEM((CHUNK,), jnp.int32),
            pltpu.VMEM((CHUNK,), jnp.int32),
            pltpu.VMEM((CHUNK, EMB), jnp.float32),
            pltpu.VMEM((CHUNK, EMB), jnp.float32),
            pltpu.SemaphoreType.DMA,
            pltpu.SemaphoreType.DMA,
            pltpu.SemaphoreType.DMA,
            pltpu.SemaphoreType.DMA,
            pltpu.SemaphoreType.DMA,
            pltpu.SemaphoreType.DMA,
        ],
        compiler_params=pltpu.CompilerParams(use_tc_tiling_on_sc=False),
    )(table, idx)


def kernel(table, post, resp):
    idx = jnp.concatenate(
        [post.reshape(-1), resp.reshape(-1)]).astype(jnp.int32)
    rows = _embed_gather(table, idx)
    return rows.reshape(2, B, L, EMB)


# speedup vs baseline: 1.4314x; 1.4279x over previous
import jax
import jax.numpy as jnp
from jax import lax
from jax.experimental import pallas as pl
from jax.experimental.pallas import tpu as pltpu
from jax.experimental.pallas import tpu_sc as plsc

VOCAB = 1000000
EMB = 32
B = 4096
L = 50

NC = 2
NS = 16
HALF = B * L
ROWS_PER_W = HALF // NS
CHUNK = 1280
NCHUNK = ROWS_PER_W // CHUNK
NBUF = 2
GW = CHUNK * EMB // 128


def _gather_body(table_hbm, ipost_hbm, iresp_hbm, out_hbm,
                 idx_v0, idx_v1, rows_v0, rows_v1,
                 si0, si1, sg0, sg1, so0, so1):
    c = lax.axis_index("c")
    s = lax.axis_index("s")
    ids = (idx_v0, idx_v1)
    rows = (rows_v0, rows_v1)
    si = (si0, si1)
    sg = (sg0, sg1)
    so = (so0, so1)

    def run(idx_hbm):
        base = s * ROWS_PER_W
        base2 = c * HALF + s * ROWS_PER_W

        def out_at(k):
            return out_hbm.at[pl.ds(base2 + k * CHUNK, CHUNK)]

        def idx_at(k):
            return idx_hbm.at[pl.ds(base + k * CHUNK, CHUNK)]

        for k in range(min(NBUF, NCHUNK)):
            pltpu.async_copy(idx_at(k), ids[k % NBUF], si[k % NBUF])

        for k in range(NCHUNK):
            b = k % NBUF
            pltpu.make_async_copy(idx_at(k), ids[b], si[b]).wait()
            if k >= NBUF:
                pltpu.make_async_copy(rows[b], out_at(k - NBUF), so[b]).wait()
            pltpu.async_copy(table_hbm.at[ids[b]], rows[b], sg[b]).wait()
            pltpu.async_copy(rows[b], out_at(k), so[b])
            if k + NBUF < NCHUNK:
                pltpu.async_copy(idx_at(k + NBUF), ids[b], si[b])

        for k in range(max(NCHUNK - NBUF, 0), NCHUNK):
            b = k % NBUF
            pltpu.make_async_copy(rows[b], out_at(k), so[b]).wait()

    @pl.when(c == 0)
    def _():
        run(ipost_hbm)

    @pl.when(c == 1)
    def _():
        run(iresp_hbm)


@jax.jit
def _embed_gather(table, ipost, iresp):
    mesh = plsc.VectorSubcoreMesh(core_axis_name="c", subcore_axis_name="s")
    return pl.kernel(
        _gather_body,
        out_type=jax.ShapeDtypeStruct((2 * HALF, EMB), jnp.float32),
        mesh=mesh,
        scratch_types=[
            pltpu.VMEM((CHUNK,), jnp.int32),
            pltpu.VMEM((CHUNK,), jnp.int32),
            pltpu.VMEM((CHUNK, EMB), jnp.float32),
            pltpu.VMEM((CHUNK, EMB), jnp.float32),
            pltpu.SemaphoreType.DMA,
            pltpu.SemaphoreType.DMA,
            pltpu.SemaphoreType.DMA,
            pltpu.SemaphoreType.DMA,
            pltpu.SemaphoreType.DMA,
            pltpu.SemaphoreType.DMA,
        ],
        compiler_params=pltpu.CompilerParams(use_tc_tiling_on_sc=False),
    )(table, ipost, iresp)


def kernel(table, post, resp):
    ipost = post.T.reshape(-1)
    iresp = resp.T.reshape(-1)
    g = _embed_gather(table, ipost, iresp)
    return g.reshape(2, L, B, EMB).transpose(0, 2, 1, 3)


# speedup vs baseline: 2.0642x; 1.4421x over previous
import jax
import jax.numpy as jnp
from jax import lax
from jax.experimental import pallas as pl
from jax.experimental.pallas import tpu as pltpu
from jax.experimental.pallas import tpu_sc as plsc

VOCAB = 1000000
EMB = 32
B = 4096
L = 50

BK = 2048
NPART = 4
NBLK = 123
Q = NBLK * BK
TPAD = NPART * Q

NC = 2
NS = 16
HALF = B * L
ROWS_PER_W = HALF // NS
CHUNK = 1280
NCHUNK = ROWS_PER_W // CHUNK
NBUF = 2


LASTB = VOCAB // BK
TAILK = LASTB - 3 * NBLK
TAIL0 = LASTB * BK


def _relayout_body(x0, x1, x2, x3, xt, y_ref):
    k = pl.program_id(0)
    x3v = jnp.where(k >= TAILK, xt[...], x3[...])
    y_ref[...] = jnp.concatenate(
        [x0[...].T, x1[...].T, x2[...].T, x3v.T], axis=1)


def _relayout_table(table_t, tail_pad):
    def in_spec(a):
        if a < 3:
            return pl.BlockSpec((EMB, BK), lambda k, a=a: (0, NBLK * a + k))
        return pl.BlockSpec(
            (EMB, BK), lambda k: (0, jnp.minimum(3 * NBLK + k, LASTB - 1)))

    tail_spec = pl.BlockSpec(
        (EMB, BK), lambda k: (0, jnp.maximum(k - TAILK, 0)))
    return pl.pallas_call(
        _relayout_body,
        grid=(NBLK,),
        in_specs=[in_spec(a) for a in range(NPART)] + [tail_spec],
        out_specs=pl.BlockSpec((BK, 128), lambda k: (k, 0)),
        out_shape=jax.ShapeDtypeStruct((Q, 128), jnp.float32),
    )(table_t, table_t, table_t, table_t, tail_pad)


def _gather_body(table_hbm, ipost_hbm, iresp_hbm, out_hbm,
                 idx_v0, idx_v1, rows_v0, rows_v1,
                 si0, si1, sg0, sg1, so0, so1):
    c = lax.axis_index("c")
    s = lax.axis_index("s")
    ids = (idx_v0, idx_v1)
    rows = (rows_v0, rows_v1)
    si = (si0, si1)
    sg = (sg0, sg1)
    so = (so0, so1)

    def run(idx_hbm):
        base = s * ROWS_PER_W
        base2 = c * HALF + s * ROWS_PER_W

        def out_at(k):
            return out_hbm.at[pl.ds(base2 + k * CHUNK, CHUNK)]

        def idx_at(k):
            return idx_hbm.at[pl.ds(base + k * CHUNK, CHUNK)]

        for k in range(min(NBUF, NCHUNK)):
            pltpu.async_copy(idx_at(k), ids[k % NBUF], si[k % NBUF])

        for k in range(NCHUNK):
            b = k % NBUF
            pltpu.make_async_copy(idx_at(k), ids[b], si[b]).wait()
            if k >= NBUF:
                pltpu.make_async_copy(rows[b], out_at(k - NBUF), so[b]).wait()
            pltpu.async_copy(table_hbm.at[ids[b]], rows[b], sg[b]).wait()
            pltpu.async_copy(rows[b], out_at(k), so[b])
            if k + NBUF < NCHUNK:
                pltpu.async_copy(idx_at(k + NBUF), ids[b], si[b])

        for k in range(max(NCHUNK - NBUF, 0), NCHUNK):
            b = k % NBUF
            pltpu.make_async_copy(rows[b], out_at(k), so[b]).wait()

    @pl.when(c == 0)
    def _():
        run(ipost_hbm)

    @pl.when(c == 1)
    def _():
        run(iresp_hbm)


def _embed_gather(table_lin, ipost, iresp):
    mesh = plsc.VectorSubcoreMesh(core_axis_name="c", subcore_axis_name="s")
    return pl.kernel(
        _gather_body,
        out_type=jax.ShapeDtypeStruct((2 * HALF, EMB), jnp.float32),
        mesh=mesh,
        scratch_types=[
            pltpu.VMEM((CHUNK,), jnp.int32),
            pltpu.VMEM((CHUNK,), jnp.int32),
            pltpu.VMEM((CHUNK, EMB), jnp.float32),
            pltpu.VMEM((CHUNK, EMB), jnp.float32),
            pltpu.SemaphoreType.DMA,
            pltpu.SemaphoreType.DMA,
            pltpu.SemaphoreType.DMA,
            pltpu.SemaphoreType.DMA,
            pltpu.SemaphoreType.DMA,
            pltpu.SemaphoreType.DMA,
        ],
        compiler_params=pltpu.CompilerParams(use_tc_tiling_on_sc=False),
    )(table_lin, ipost, iresp)


@jax.jit
def _pointer_embedding(table, post, resp):
    table_t = table.T
    tail_pad = jnp.pad(
        table_t[:, TAIL0:], ((0, 0), (0, (NBLK - TAILK) * BK - (VOCAB - TAIL0))))
    table_lin = _relayout_table(table_t, tail_pad).reshape(TPAD, EMB)

    def to_lin_idx(x):
        v = x.T.reshape(-1)
        a = v // Q
        return 4 * (v - a * Q) + a

    g = _embed_gather(table_lin, to_lin_idx(post), to_lin_idx(resp))
    return g.reshape(2, L, B, EMB).transpose(0, 2, 1, 3)


def kernel(table, post, resp):
    return _pointer_embedding(table, post, resp)
